# baseline (speedup 1.0000x reference)
import functools
import jax
import jax.numpy as jnp
from jax import lax
from jax.experimental import pallas as pl
from jax.experimental.pallas import tpu as pltpu
from jax.experimental.pallas import tpu_sc as plsc

LANES = 16
N_CORES = 2
N_SUBCORES = 16
NW = N_CORES * N_SUBCORES
TCW = 128
STG = 7
HCAP = 784
SCAP = 112


@jax.jit
def _run(user_ids, movie_ids, uf_t, mf_t):
    B = user_ids.shape[0]
    F = uf_t.shape[0]
    V = uf_t.shape[1]
    NT = (V + TCW - 1) // TCW
    NT_FULL = V // TCW
    n_full = NT_FULL * TCW
    tail_w = V - n_full
    TPW = (NT + NW - 1) // NW
    NSTG = (TPW + STG - 1) // STG
    WIN = STG * TCW
    last_win = (NT_FULL - STG) * TCW

    mesh = plsc.VectorSubcoreMesh(core_axis_name="c", subcore_axis_name="s")

    @functools.partial(
        pl.kernel,
        mesh=mesh,
        compiler_params=pltpu.CompilerParams(needs_layout_passes=False),
        out_type=jax.ShapeDtypeStruct((B + 1, F), jnp.float32),
        scratch_types=[
            pltpu.VMEM((B,), jnp.int32),
            pltpu.VMEM((HCAP,), jnp.int32),
            pltpu.VMEM((HCAP,), jnp.int32),
            pltpu.VMEM((SCAP,), jnp.int32),
            pltpu.VMEM((SCAP,), jnp.int32),
            pltpu.VMEM((F, WIN), jnp.float32),
            pltpu.VMEM((F, tail_w), jnp.float32),
            pltpu.VMEM((LANES, F), jnp.float32),
            pltpu.SemaphoreType.DMA,
        ],
    )
    def k1(ids_in_hbm, tab_hbm, rows_hbm,
           ids_v, hb_v, hid_v, sb_v, sid_v, win_v, tail_v, row_v, sem):
        wid = lax.axis_index("s") * N_CORES + lax.axis_index("c")
        lo = wid * TPW
        hi = jnp.minimum(lo + TPW, NT)

        lane = lax.broadcasted_iota(jnp.int32, (LANES,), 0)

        def one_table(ids_hbm, tab_hbm, rows_hbm):
            pltpu.sync_copy(ids_hbm, ids_v)
            pltpu.sync_copy(tab_hbm.at[:, pl.ds(n_full, tail_w)], tail_v)

            def scan(g, cnt):
                idv = ids_v[pl.ds(g * LANES, LANES)]
                t = lax.shift_right_logical(idv, 7)
                m = jnp.logical_and(t >= lo, t < hi)
                plsc.store_compressed(
                    hb_v.at[pl.ds(cnt, LANES)], g * LANES + lane, mask=m)
                plsc.store_compressed(
                    hid_v.at[pl.ds(cnt, LANES)], idv, mask=m)
                npc = plsc.all_reduce_population_count(m)
                return cnt + npc[0]

            cnt = lax.fori_loop(0, B // LANES, scan, 0)
            ngrp = lax.shift_right_logical(cnt + LANES - 1, 4)

            def stage(s, carry):
                stage_lo = lo + s * STG
                stage_hi = jnp.minimum(stage_lo + STG, hi)
                wbase = jnp.minimum(stage_lo * TCW, last_win)
                wbase = pl.multiple_of(wbase, TCW)
                pltpu.sync_copy(tab_hbm.at[:, pl.ds(wbase, WIN)], win_v)

                def compact(h, c2):
                    hb = hb_v[pl.ds(h * LANES, LANES)]
                    hid = hid_v[pl.ds(h * LANES, LANES)]
                    t = lax.shift_right_logical(hid, 7)
                    m = jnp.logical_and(
                        jnp.logical_and(t >= stage_lo, t < stage_hi),
                        (h * LANES + lane) < cnt)
                    plsc.store_compressed(
                        sb_v.at[pl.ds(c2, LANES)], hb, mask=m)
                    plsc.store_compressed(
                        sid_v.at[pl.ds(c2, LANES)], hid, mask=m)
                    npc = plsc.all_reduce_population_count(m)
                    return c2 + npc[0]

                cnt2 = lax.fori_loop(0, ngrp, compact, 0)
                ngrp2 = lax.shift_right_logical(cnt2 + LANES - 1, 4)

                def extract(h, carry2):
                    sb = sb_v[pl.ds(h * LANES, LANES)]
                    sid = sid_v[pl.ds(h * LANES, LANES)]
                    valid = (h * LANES + lane) < cnt2
                    bsel = jnp.where(valid, sb, B)
                    col = jnp.clip(sid - wbase, 0, WIN - 1)
                    tcol = jnp.bitwise_and(sid - n_full, tail_w - 1)
                    is_tail = sid >= n_full
                    for f in range(F):
                        fv = jnp.full((LANES,), f, jnp.int32)
                        v = plsc.load_gather(win_v, [fv, col])
                        vt = plsc.load_gather(tail_v, [fv, tcol])
                        val = jnp.where(is_tail, vt, v)
                        plsc.store_scatter(row_v, [lane, fv], val)
                    cps = []
                    for j in range(LANES):
                        cps.append(pltpu.async_copy(
                            row_v.at[j], rows_hbm.at[bsel[j]], sem))
                    for cp in cps:
                        cp.wait()
                    return carry2

                lax.fori_loop(0, ngrp2, extract, 0)
                return carry

            lax.fori_loop(0, NSTG, stage, 0)

        one_table(ids_in_hbm, tab_hbm, rows_hbm)

    def _dot_body(u_ref, m_ref, o_ref):
        o_ref[...] = jnp.sum(u_ref[...] * m_ref[...], axis=1, keepdims=True)

    def dot_tc(urows, mrows):
        Bp = urows.shape[0]
        blk = 1024
        grid = (Bp + blk - 1) // blk
        return pl.pallas_call(
            _dot_body,
            grid=(grid,),
            in_specs=[pl.BlockSpec((blk, F), lambda i: (i, 0)),
                      pl.BlockSpec((blk, F), lambda i: (i, 0))],
            out_specs=pl.BlockSpec((blk, 1), lambda i: (i, 0)),
            out_shape=jax.ShapeDtypeStruct((Bp, 1), jnp.float32),
        )(urows, mrows)

    urows = k1(user_ids, uf_t)
    mrows = k1(movie_ids, mf_t)
    return dot_tc(urows, mrows)[:B]


def kernel(user_ids, movie_ids, user_factors, movie_factors):
    return _run(user_ids.astype(jnp.int32), movie_ids.astype(jnp.int32),
                user_factors.T, movie_factors.T)


# speedup vs baseline: 1.0007x; 1.0007x over previous
import functools
import jax
import jax.numpy as jnp
from jax import lax
from jax.experimental import pallas as pl
from jax.experimental.pallas import tpu as pltpu
from jax.experimental.pallas import tpu_sc as plsc

LANES = 16
N_CORES = 2
N_SUBCORES = 16
NW = N_CORES * N_SUBCORES
TCW = 128
STG = 7
HCAP = 784
SCAP = 112


@jax.jit
def _run(user_ids, movie_ids, uf_t, mf_t):
    B = user_ids.shape[0]
    F = uf_t.shape[0]
    V = uf_t.shape[1]
    NT = (V + TCW - 1) // TCW
    NT_FULL = V // TCW
    n_full = NT_FULL * TCW
    tail_w = V - n_full
    TPW = (NT + NW - 1) // NW
    NSTG = (TPW + STG - 1) // STG
    WIN = STG * TCW
    last_win = (NT_FULL - STG) * TCW

    mesh = plsc.VectorSubcoreMesh(core_axis_name="c", subcore_axis_name="s")

    @functools.partial(
        pl.kernel,
        mesh=mesh,
        compiler_params=pltpu.CompilerParams(needs_layout_passes=False),
        out_type=jax.ShapeDtypeStruct((B + 1, F), jnp.float32),
        scratch_types=[
            pltpu.VMEM((B,), jnp.int32),
            pltpu.VMEM((HCAP,), jnp.int32),
            pltpu.VMEM((HCAP,), jnp.int32),
            pltpu.VMEM((SCAP,), jnp.int32),
            pltpu.VMEM((SCAP,), jnp.int32),
            pltpu.VMEM((2, F, WIN), jnp.float32),
            pltpu.VMEM((F, tail_w), jnp.float32),
            pltpu.VMEM((SCAP, F), jnp.float32),
            pltpu.SemaphoreType.DMA,
            pltpu.SemaphoreType.DMA,
        ],
    )
    def k1(ids_in_hbm, tab_hbm, rows_hbm,
           ids_v, hb_v, hid_v, sb_v, sid_v, win_v, tail_v, row_v,
           sem, sem_w):
        wid = lax.axis_index("s") * N_CORES + lax.axis_index("c")
        lo = wid * TPW
        hi = jnp.minimum(lo + TPW, NT)

        lane = lax.broadcasted_iota(jnp.int32, (LANES,), 0)

        def one_table(ids_hbm, tab_hbm, rows_hbm):
            pltpu.sync_copy(ids_hbm, ids_v)
            pltpu.sync_copy(tab_hbm.at[:, pl.ds(n_full, tail_w)], tail_v)

            def scan(g, cnt):
                idv = ids_v[pl.ds(g * LANES, LANES)]
                t = lax.shift_right_logical(idv, 7)
                m = jnp.logical_and(t >= lo, t < hi)
                plsc.store_compressed(
                    hb_v.at[pl.ds(cnt, LANES)], g * LANES + lane, mask=m)
                plsc.store_compressed(
                    hid_v.at[pl.ds(cnt, LANES)], idv, mask=m)
                npc = plsc.all_reduce_population_count(m)
                return cnt + npc[0]

            cnt = lax.fori_loop(0, B // LANES, scan, 0)
            ngrp = lax.shift_right_logical(cnt + LANES - 1, 4)

            def wbase_of(s):
                sl = lo + s * STG
                return pl.multiple_of(
                    jnp.minimum(sl * TCW, last_win), TCW)

            pltpu.async_copy(
                tab_hbm.at[:, pl.ds(wbase_of(0), WIN)], win_v.at[0], sem_w)

            def stage(s, carry):
                stage_lo = lo + s * STG
                stage_hi = jnp.minimum(stage_lo + STG, hi)
                wbase = wbase_of(s)
                cur = jnp.bitwise_and(s, 1)
                pltpu.make_async_copy(
                    tab_hbm.at[:, pl.ds(0, WIN)], win_v.at[0], sem_w).wait()
                nxt_s = jnp.minimum(s + 1, NSTG - 1)
                pltpu.async_copy(
                    tab_hbm.at[:, pl.ds(wbase_of(nxt_s), WIN)],
                    win_v.at[1 - cur], sem_w)

                def compact(h, c2):
                    hb = hb_v[pl.ds(h * LANES, LANES)]
                    hid = hid_v[pl.ds(h * LANES, LANES)]
                    t = lax.shift_right_logical(hid, 7)
                    m = jnp.logical_and(
                        jnp.logical_and(t >= stage_lo, t < stage_hi),
                        (h * LANES + lane) < cnt)
                    plsc.store_compressed(
                        sb_v.at[pl.ds(c2, LANES)], hb, mask=m)
                    plsc.store_compressed(
                        sid_v.at[pl.ds(c2, LANES)], hid, mask=m)
                    npc = plsc.all_reduce_population_count(m)
                    return c2 + npc[0]

                cnt2 = lax.fori_loop(0, ngrp, compact, 0)
                ngrp2 = lax.shift_right_logical(cnt2 + LANES - 1, 4)

                def extract(h, carry2):
                    sb = sb_v[pl.ds(h * LANES, LANES)]
                    sid = sid_v[pl.ds(h * LANES, LANES)]
                    valid = (h * LANES + lane) < cnt2
                    bsel = jnp.where(valid, sb, B)
                    slot = h * LANES + lane
                    col = jnp.clip(sid - wbase, 0, WIN - 1)
                    tcol = jnp.bitwise_and(sid - n_full, tail_w - 1)
                    is_tail = sid >= n_full
                    cv = jnp.full((LANES,), cur, jnp.int32)
                    for f in range(F):
                        fv = jnp.full((LANES,), f, jnp.int32)
                        v = plsc.load_gather(win_v, [cv, fv, col])
                        vt = plsc.load_gather(tail_v, [fv, tcol])
                        val = jnp.where(is_tail, vt, v)
                        plsc.store_scatter(row_v, [slot, fv], val)
                    for j in range(LANES):
                        pltpu.async_copy(
                            row_v.at[pl.ds(h * LANES + j, 1)],
                            rows_hbm.at[pl.ds(bsel[j], 1)], sem)
                    return carry2

                lax.fori_loop(0, ngrp2, extract, 0)

                def drain_rows(h, carry3):
                    pltpu.make_async_copy(
                        rows_hbm.at[pl.ds(0, 1)],
                        row_v.at[pl.ds(0, 1)], sem).wait()
                    return carry3

                lax.fori_loop(0, cnt2 + (LANES - jnp.bitwise_and(
                    cnt2, LANES - 1)) % LANES, drain_rows, 0)
                return carry

            lax.fori_loop(0, NSTG, stage, 0)
            pltpu.make_async_copy(
                tab_hbm.at[:, pl.ds(0, WIN)], win_v.at[0], sem_w).wait()

        one_table(ids_in_hbm, tab_hbm, rows_hbm)

    def _dot_body(u_ref, m_ref, o_ref):
        o_ref[...] = jnp.sum(u_ref[...] * m_ref[...], axis=1, keepdims=True)

    def dot_tc(urows, mrows):
        Bp = urows.shape[0]
        blk = 1024
        grid = (Bp + blk - 1) // blk
        return pl.pallas_call(
            _dot_body,
            grid=(grid,),
            in_specs=[pl.BlockSpec((blk, F), lambda i: (i, 0)),
                      pl.BlockSpec((blk, F), lambda i: (i, 0))],
            out_specs=pl.BlockSpec((blk, 1), lambda i: (i, 0)),
            out_shape=jax.ShapeDtypeStruct((Bp, 1), jnp.float32),
        )(urows, mrows)

    urows = k1(user_ids, uf_t)
    mrows = k1(movie_ids, mf_t)
    return dot_tc(urows, mrows)[:B]


def kernel(user_ids, movie_ids, user_factors, movie_factors):
    return _run(user_ids.astype(jnp.int32), movie_ids.astype(jnp.int32),
                user_factors.T, movie_factors.T)


# speedup vs baseline: 3.3668x; 3.3645x over previous
import functools
import jax
import jax.numpy as jnp
from jax import lax
from jax.experimental import pallas as pl
from jax.experimental.pallas import tpu as pltpu
from jax.experimental.pallas import tpu_sc as plsc

LANES = 16
N_CORES = 2
N_SUBCORES = 16
GRP = 8
TCW = 128
LAST_BASE = 0


@jax.jit
def _run(user_ids, movie_ids, uf_t, mf_t):
    B = user_ids.shape[0]
    F = uf_t.shape[0]
    V = uf_t.shape[1]
    NW = N_CORES * N_SUBCORES
    BPW = B // NW

    n_full = (V // TCW) * TCW
    last_base = n_full - TCW
    tail_w = V - n_full

    mesh = plsc.VectorSubcoreMesh(core_axis_name="c", subcore_axis_name="s")

    @functools.partial(
        pl.kernel,
        mesh=mesh,
        compiler_params=pltpu.CompilerParams(needs_layout_passes=False),
        out_type=jax.ShapeDtypeStruct((B,), jnp.float32),
        scratch_types=[
            pltpu.VMEM((BPW,), jnp.int32),
            pltpu.VMEM((BPW,), jnp.int32),
            pltpu.VMEM((GRP, F, TCW), jnp.float32),
            pltpu.VMEM((GRP, F, TCW), jnp.float32),
            pltpu.VMEM((F, tail_w), jnp.float32),
            pltpu.VMEM((F, tail_w), jnp.float32),
            pltpu.VMEM((BPW,), jnp.float32),
            pltpu.SemaphoreType.DMA,
            pltpu.SemaphoreType.DMA,
        ],
    )
    def sc_kernel(uids_hbm, mids_hbm, uf_hbm, mf_hbm, out_hbm,
                  uidx_v, midx_v, uring_v, mring_v, utail_v, mtail_v,
                  out_v, sem_u, sem_m):
        wid = lax.axis_index("s") * N_CORES + lax.axis_index("c")
        base = wid * BPW

        pltpu.sync_copy(uids_hbm.at[pl.ds(base, BPW)], uidx_v)
        pltpu.sync_copy(mids_hbm.at[pl.ds(base, BPW)], midx_v)
        pltpu.sync_copy(uf_hbm.at[:, pl.ds(n_full, tail_w)], utail_v)
        pltpu.sync_copy(mf_hbm.at[:, pl.ds(n_full, tail_w)], mtail_v)

        lane = lax.broadcasted_iota(jnp.int32, (LANES,), 0)
        slot = jnp.bitwise_and(lane, GRP - 1)

        def pair(p, carry):
            uvec = uidx_v[pl.ds(p * 2 * GRP, LANES)]
            mvec = midx_v[pl.ds(p * 2 * GRP, LANES)]
            ubase = jnp.minimum(
                jnp.bitwise_and(uvec, ~(TCW - 1)), last_base)
            mbase = jnp.minimum(
                jnp.bitwise_and(mvec, ~(TCW - 1)), last_base)
            uc = uvec - ubase
            mc = mvec - mbase
            ucl = jnp.minimum(uc, TCW - 1)
            mcl = jnp.minimum(mc, TCW - 1)
            uct = jnp.bitwise_and(uvec - n_full, tail_w - 1)
            mct = jnp.bitwise_and(mvec - n_full, tail_w - 1)
            u_is_tail = uvec >= n_full
            m_is_tail = mvec >= n_full

            def phase(lo):
                cps = []
                for j in range(GRP):
                    cps.append(pltpu.async_copy(
                        uf_hbm.at[:, pl.ds(pl.multiple_of(ubase[lo + j], TCW), TCW)],
                        uring_v.at[j], sem_u))
                    cps.append(pltpu.async_copy(
                        mf_hbm.at[:, pl.ds(pl.multiple_of(mbase[lo + j], TCW), TCW)],
                        mring_v.at[j], sem_m))
                for cp in cps:
                    cp.wait()
                acc = jnp.zeros((LANES,), jnp.float32)
                for f in range(F):
                    fvec = jnp.full((LANES,), f, jnp.int32)
                    u = plsc.load_gather(uring_v, [slot, fvec, ucl])
                    m = plsc.load_gather(mring_v, [slot, fvec, mcl])
                    ut = plsc.load_gather(utail_v, [fvec, uct])
                    mt = plsc.load_gather(mtail_v, [fvec, mct])
                    uv = jnp.where(u_is_tail, ut, u)
                    mv = jnp.where(m_is_tail, mt, m)
                    acc = acc + uv * mv
                return acc

            acc_lo = phase(0)
            acc_hi = phase(GRP)
            res = jnp.where(lane < GRP, acc_lo, acc_hi)
            out_v[pl.ds(p * 2 * GRP, LANES)] = res
            return carry

        lax.fori_loop(0, BPW // (2 * GRP), pair, 0)

        pltpu.sync_copy(out_v, out_hbm.at[pl.ds(base, BPW)])

    return sc_kernel(user_ids, movie_ids, uf_t, mf_t)


def kernel(user_ids, movie_ids, user_factors, movie_factors):
    out = _run(user_ids.astype(jnp.int32), movie_ids.astype(jnp.int32),
               user_factors.T, movie_factors.T)
    return out.reshape(-1, 1)


# speedup vs baseline: 3.6335x; 1.0792x over previous
import functools
import jax
import jax.numpy as jnp
from jax import lax
from jax.experimental import pallas as pl
from jax.experimental.pallas import tpu as pltpu
from jax.experimental.pallas import tpu_sc as plsc

LANES = 16
N_CORES = 2
N_SUBCORES = 16
U = 4
TCW = 128


@jax.jit
def _run(user_ids, movie_ids, uf_t, mf_t):
    B = user_ids.shape[0]
    F = uf_t.shape[0]
    V = uf_t.shape[1]
    NW = N_CORES * N_SUBCORES
    BPW = B // NW
    NQ = BPW // LANES

    n_full = (V // TCW) * TCW
    last_base = n_full - TCW
    tail_w = V - n_full

    mesh = plsc.VectorSubcoreMesh(core_axis_name="c", subcore_axis_name="s")

    @functools.partial(
        pl.kernel,
        mesh=mesh,
        compiler_params=pltpu.CompilerParams(needs_layout_passes=False),
        out_type=jax.ShapeDtypeStruct((B,), jnp.float32),
        scratch_types=[
            pltpu.VMEM((BPW,), jnp.int32),
            pltpu.VMEM((BPW,), jnp.int32),
            pltpu.VMEM((2, U, F, TCW), jnp.float32),
            pltpu.VMEM((2, U, F, TCW), jnp.float32),
            pltpu.VMEM((F, tail_w), jnp.float32),
            pltpu.VMEM((F, tail_w), jnp.float32),
            pltpu.VMEM((BPW,), jnp.float32),
            pltpu.SemaphoreType.DMA,
            pltpu.SemaphoreType.DMA,
            pltpu.SemaphoreType.DMA,
            pltpu.SemaphoreType.DMA,
        ],
    )
    def sc_kernel(uids_hbm, mids_hbm, uf_hbm, mf_hbm, out_hbm,
                  uidx_v, midx_v, uring_v, mring_v, utail_v, mtail_v,
                  out_v, sem_u0, sem_u1, sem_m0, sem_m1):
        wid = lax.axis_index("s") * N_CORES + lax.axis_index("c")
        base = wid * BPW

        sem_u = (sem_u0, sem_u1)
        sem_m = (sem_m0, sem_m1)
        pltpu.sync_copy(uids_hbm.at[pl.ds(base, BPW)], uidx_v)
        pltpu.sync_copy(mids_hbm.at[pl.ds(base, BPW)], midx_v)
        pltpu.sync_copy(uf_hbm.at[:, pl.ds(n_full, tail_w)], utail_v)
        pltpu.sync_copy(mf_hbm.at[:, pl.ds(n_full, tail_w)], mtail_v)

        lane = lax.broadcasted_iota(jnp.int32, (LANES,), 0)
        slot4 = jnp.bitwise_and(lane, U - 1)

        def bases_at(off):
            uvec = uidx_v[pl.ds(off, LANES)]
            mvec = midx_v[pl.ds(off, LANES)]
            ubase = jnp.minimum(jnp.bitwise_and(uvec, ~(TCW - 1)), last_base)
            mbase = jnp.minimum(jnp.bitwise_and(mvec, ~(TCW - 1)), last_base)
            return uvec, mvec, ubase, mbase

        def fire(s, lo, ubase, mbase):
            for j in range(U):
                pltpu.async_copy(
                    uf_hbm.at[:, pl.ds(pl.multiple_of(ubase[lo + j], TCW),
                                       TCW)],
                    uring_v.at[s, j], sem_u[s])
                pltpu.async_copy(
                    mf_hbm.at[:, pl.ds(pl.multiple_of(mbase[lo + j], TCW),
                                       TCW)],
                    mring_v.at[s, j], sem_m[s])

        def wait_unit(s):
            for j in range(U):
                pltpu.make_async_copy(
                    uf_hbm.at[:, pl.ds(0, TCW)],
                    uring_v.at[0, j], sem_u[s]).wait()
                pltpu.make_async_copy(
                    mf_hbm.at[:, pl.ds(0, TCW)],
                    mring_v.at[0, j], sem_m[s]).wait()

        uvec0, mvec0, ubase0, mbase0 = bases_at(0)
        fire(0, 0, ubase0, mbase0)
        fire(1, U, ubase0, mbase0)

        def quad(q, carry):
            uvec, mvec, ubase, mbase = bases_at(q * LANES)
            offN = jnp.minimum((q + 1) * LANES, BPW - LANES)
            uvecN, mvecN, ubaseN, mbaseN = bases_at(offN)

            ucl = jnp.minimum(uvec - ubase, TCW - 1)
            mcl = jnp.minimum(mvec - mbase, TCW - 1)
            uct = jnp.bitwise_and(uvec - n_full, tail_w - 1)
            mct = jnp.bitwise_and(mvec - n_full, tail_w - 1)
            u_is_tail = uvec >= n_full
            m_is_tail = mvec >= n_full

            def unitdot(s):
                sv = jnp.full((LANES,), s, jnp.int32)
                acc = jnp.zeros((LANES,), jnp.float32)
                for f in range(F):
                    fv = jnp.full((LANES,), f, jnp.int32)
                    u = plsc.load_gather(uring_v, [sv, slot4, fv, ucl])
                    m = plsc.load_gather(mring_v, [sv, slot4, fv, mcl])
                    ut = plsc.load_gather(utail_v, [fv, uct])
                    mt = plsc.load_gather(mtail_v, [fv, mct])
                    uv = jnp.where(u_is_tail, ut, u)
                    mv = jnp.where(m_is_tail, mt, m)
                    acc = acc + uv * mv
                return acc

            wait_unit(0)
            acc0 = unitdot(0)
            fire(0, 2 * U, ubase, mbase)
            wait_unit(1)
            acc1 = unitdot(1)
            fire(1, 3 * U, ubase, mbase)
            wait_unit(0)
            acc2 = unitdot(0)
            fire(0, 0, ubaseN, mbaseN)
            wait_unit(1)
            acc3 = unitdot(1)
            fire(1, U, ubaseN, mbaseN)

            res = jnp.where(
                lane < U, acc0,
                jnp.where(lane < 2 * U, acc1,
                          jnp.where(lane < 3 * U, acc2, acc3)))
            out_v[pl.ds(q * LANES, LANES)] = res
            return carry

        lax.fori_loop(0, NQ, quad, 0)
        wait_unit(0)
        wait_unit(1)

        pltpu.sync_copy(out_v, out_hbm.at[pl.ds(base, BPW)])

    return sc_kernel(user_ids, movie_ids, uf_t, mf_t)


def kernel(user_ids, movie_ids, user_factors, movie_factors):
    out = _run(user_ids.astype(jnp.int32), movie_ids.astype(jnp.int32),
               user_factors.T, movie_factors.T)
    return out.reshape(-1, 1)
